# baseline (speedup 1.0000x reference)
import functools

import jax
import jax.numpy as jnp
from jax import lax
from jax.experimental import pallas as pl
from jax.experimental.pallas import tpu as pltpu
from jax.experimental.pallas import tpu_sc as plsc

_N = 10000
_E = 320000
_D = 128
_H = 64

_NC = 2
_NS = 16
_NW = _NC * _NS

_RPT = 640
_NPAD = _NS * _RPT
_CHUNK = 128
_EPT = 10240
_NCH = _EPT // _CHUNK
_EPAD = _NW * _EPT
_W128 = 128

_mesh = plsc.VectorSubcoreMesh(
    core_axis_name="c", subcore_axis_name="s", num_cores=_NC, num_subcores=_NS
)


_K = 1
_NG = _NCH // _K


@functools.partial(
    pl.kernel,
    out_type=jax.ShapeDtypeStruct((_NC, _NPAD, _W128), jnp.float32),
    mesh=_mesh,
    scratch_types=[
        pltpu.VMEM((_NCH, _CHUNK), jnp.int32),
        pltpu.VMEM((_NCH, _CHUNK), jnp.int32),
        pltpu.VMEM((2, _K, _CHUNK, _W128), jnp.float32),
        pltpu.VMEM((2, _K, _CHUNK, _H), jnp.float32),
        pltpu.VMEM_SHARED((_NPAD, _H), jnp.float32),
        pltpu.SemaphoreType.DMA,
        pltpu.SemaphoreType.DMA,
        pltpu.SemaphoreType.DMA,
        pltpu.SemaphoreType.DMA,
    ],
    compiler_params=pltpu.CompilerParams(use_tc_tiling_on_sc=False),
)
def _sc_agg(
    g_hbm, src_hbm, dst_hbm, out_hbm, src_v, dst_v, bufs, hbufs, acc_sh,
    gsem_a, gsem_b, ssem_a, ssem_b,
):
    c = lax.axis_index("c")
    s = lax.axis_index("s")
    wid = c * _NS + s
    col0 = c * _H
    pltpu.sync_copy(src_hbm.at[wid], src_v)
    pltpu.sync_copy(dst_hbm.at[wid], dst_v)
    r0 = s * _RPT
    pltpu.sync_copy(
        g_hbm.at[pl.ds(r0, _RPT), pl.ds(col0, _H)], acc_sh.at[pl.ds(r0, _RPT)]
    )
    plsc.subcore_barrier()

    def gathers(h, grp, sem):
        for k in range(_K):
            pltpu.async_copy(g_hbm.at[src_v.at[grp * _K + k]], bufs.at[h, k], sem)

    def wait_gathers(h, grp, sem):
        for k in range(_K):
            pltpu.make_async_copy(
                g_hbm.at[src_v.at[grp * _K + k]], bufs.at[h, k], sem
            ).wait()

    def halve(h):
        for k in range(_K):

            def hrow(i, carry):
                for j4 in range(_H // 16):
                    hbufs[h, k, i, pl.ds(16 * j4, 16)] = bufs[
                        h, k, i, pl.ds(16 * j4, 16)
                    ]
                return carry

            lax.fori_loop(0, _CHUNK, hrow, 0)

    def scatters(h, grp, sem):
        for k in range(_K):
            pltpu.async_copy(
                hbufs.at[h, k], acc_sh.at[dst_v.at[grp * _K + k]], sem, add=True
            )

    def wait_scatters(h, grp, sem):
        for k in range(_K):
            pltpu.make_async_copy(
                hbufs.at[h, k], acc_sh.at[dst_v.at[grp * _K + k]], sem
            ).wait()

    gathers(0, 0, gsem_a)

    def body(j, carry):
        ga = 2 * j
        gb = 2 * j + 1
        wait_gathers(0, ga, gsem_a)
        halve(0)
        scatters(0, ga, ssem_a)

        @pl.when(j > 0)
        def _():
            wait_scatters(1, gb - 2, ssem_b)

        gathers(1, gb, gsem_b)
        wait_gathers(1, gb, gsem_b)
        halve(1)
        scatters(1, gb, ssem_b)

        @pl.when(j < _NG // 2 - 1)
        def _():
            wait_scatters(0, ga, ssem_a)
            gathers(0, ga + 2, gsem_a)

        return carry

    lax.fori_loop(0, _NG // 2, body, 0)
    wait_scatters(0, _NG - 2, ssem_a)
    wait_scatters(1, _NG - 1, ssem_b)
    plsc.subcore_barrier()
    pltpu.sync_copy(
        acc_sh.at[pl.ds(r0, _RPT)],
        out_hbm.at[c].at[pl.ds(r0, _RPT), pl.ds(0, _H)],
    )


_BLK = 512
_GRID = _NPAD // _BLK


def _dis_blk(c0, c1, i):
    deg = c0[:, 0:1] + c1[:, 0:1]
    rows = i * _BLK + lax.broadcasted_iota(jnp.int32, (_BLK, 1), 0)
    return jnp.where(rows < _N, lax.rsqrt(deg), 0.0)


def _wide(v):
    return jnp.concatenate([v, jnp.zeros((_BLK, _W128 - _H), jnp.float32)], axis=1)


def _bspec():
    return pl.BlockSpec((_BLK, _W128), lambda i: (i, 0))


def _tc_prologue_body(x_ref, w_ref, c0_ref, c1_ref, o_ref):
    i = pl.program_id(0)
    dis = _dis_blk(c0_ref[...], c1_ref[...], i)
    o_ref[...] = _wide(
        dis * jnp.dot(x_ref[...], w_ref[...], preferred_element_type=jnp.float32)
    )


_tc_prologue = pl.pallas_call(
    _tc_prologue_body,
    out_shape=jax.ShapeDtypeStruct((_NPAD, _W128), jnp.float32),
    grid=(_GRID,),
    in_specs=[
        pl.BlockSpec((_BLK, _D), lambda i: (i, 0)),
        pl.BlockSpec((_D, _H), lambda i: (0, 0)),
        _bspec(),
        _bspec(),
    ],
    out_specs=_bspec(),
)


def _tc_fuse_body(a0_ref, a1_ref, c0_ref, c1_ref, b_ref, w_ref, o_ref):
    i = pl.program_id(0)
    dis = _dis_blk(c0_ref[...], c1_ref[...], i)
    am = (a0_ref[...] + a1_ref[...])[:, :_H]
    h = dis * am + b_ref[...]
    h = jnp.maximum(h, 0.0)
    o_ref[...] = _wide(
        dis * jnp.dot(h, w_ref[...], preferred_element_type=jnp.float32)
    )


_tc_fuse = pl.pallas_call(
    _tc_fuse_body,
    out_shape=jax.ShapeDtypeStruct((_NPAD, _W128), jnp.float32),
    grid=(_GRID,),
    in_specs=[
        _bspec(),
        _bspec(),
        _bspec(),
        _bspec(),
        pl.BlockSpec((1, _H), lambda i: (0, 0)),
        pl.BlockSpec((_H, _H), lambda i: (0, 0)),
    ],
    out_specs=_bspec(),
)


def _tc_final_body(a0_ref, a1_ref, c0_ref, c1_ref, b_ref, w_ref, bo_ref, o_ref):
    i = pl.program_id(0)
    dis = _dis_blk(c0_ref[...], c1_ref[...], i)
    am = (a0_ref[...] + a1_ref[...])[:, :_H]
    h = dis * am + b_ref[...]
    o_ref[...] = (
        jnp.dot(h, w_ref[...], preferred_element_type=jnp.float32) + bo_ref[...]
    )


_tc_final = pl.pallas_call(
    _tc_final_body,
    out_shape=jax.ShapeDtypeStruct((_NPAD, _D), jnp.float32),
    grid=(_GRID,),
    in_specs=[
        _bspec(),
        _bspec(),
        _bspec(),
        _bspec(),
        pl.BlockSpec((1, _H), lambda i: (0, 0)),
        pl.BlockSpec((_H, _D), lambda i: (0, 0)),
        pl.BlockSpec((1, _D), lambda i: (0, 0)),
    ],
    out_specs=pl.BlockSpec((_BLK, _D), lambda i: (i, 0)),
)


def kernel(x, edge_index, W0, b0, W1, b1, W2, b2, W3, b3, Wout, bout):
    src = edge_index[0].astype(jnp.int32)
    dst = edge_index[1].astype(jnp.int32)
    npad = _EPAD - _E
    srcp = jnp.concatenate([src, jnp.full((npad,), _N, jnp.int32)]).reshape(
        _NW, _NCH, _CHUNK
    )
    dstp = jnp.concatenate([dst, jnp.full((npad,), _N, jnp.int32)]).reshape(
        _NW, _NCH, _CHUNK
    )
    xp = jnp.pad(x, ((0, _NPAD - _N), (0, 0)))

    ones_wide = jnp.concatenate(
        [jnp.ones((_NPAD, _H), jnp.float32), jnp.zeros((_NPAD, _H), jnp.float32)],
        axis=1,
    )
    cacc = _sc_agg(ones_wide, srcp, dstp)
    c0, c1 = cacc[0], cacc[1]

    g = _tc_prologue(xp, W0, c0, c1)
    for bb, ww in ((b0, W1), (b1, W2), (b2, W3)):
        acc = _sc_agg(g, srcp, dstp)
        g = _tc_fuse(acc[0], acc[1], c0, c1, bb.reshape(1, _H), ww)
    acc = _sc_agg(g, srcp, dstp)
    out = _tc_final(
        acc[0], acc[1], c0, c1, b3.reshape(1, _H), Wout, bout.reshape(1, _D)
    )
    return out[:_N]


# speedup vs baseline: 1.0136x; 1.0136x over previous
import functools

import jax
import jax.numpy as jnp
from jax import lax
from jax.experimental import pallas as pl
from jax.experimental.pallas import tpu as pltpu
from jax.experimental.pallas import tpu_sc as plsc

_N = 10000
_E = 320000
_D = 128
_H = 64

_NC = 2
_NS = 16
_NW = _NC * _NS

_RPT = 640
_NPAD = _NS * _RPT
_CHUNK = 128
_EPT = 10240
_NCH = _EPT // _CHUNK
_EPAD = _NW * _EPT
_W128 = 128

_mesh = plsc.VectorSubcoreMesh(
    core_axis_name="c", subcore_axis_name="s", num_cores=_NC, num_subcores=_NS
)


_K = 1
_NG = _NCH // _K


@functools.partial(
    pl.kernel,
    out_type=jax.ShapeDtypeStruct((_NC, _NPAD, _W128), jnp.float32),
    mesh=_mesh,
    scratch_types=[
        pltpu.VMEM((_NCH, _CHUNK), jnp.int32),
        pltpu.VMEM((_NCH, _CHUNK), jnp.int32),
        pltpu.VMEM((2, _K, _CHUNK, _W128), jnp.float32),
        pltpu.VMEM((2, _K, _CHUNK, _H), jnp.float32),
        pltpu.VMEM_SHARED((_NPAD, _H), jnp.float32),
        pltpu.SemaphoreType.DMA,
        pltpu.SemaphoreType.DMA,
        pltpu.SemaphoreType.DMA,
        pltpu.SemaphoreType.DMA,
    ],
    compiler_params=pltpu.CompilerParams(use_tc_tiling_on_sc=False),
)
def _sc_agg(
    g_hbm, src_hbm, dst_hbm, out_hbm, src_v, dst_v, bufs, hbufs, acc_sh,
    gsem_a, gsem_b, ssem_a, ssem_b,
):
    c = lax.axis_index("c")
    s = lax.axis_index("s")
    wid = c * _NS + s
    col0 = c * _H
    pltpu.sync_copy(src_hbm.at[wid], src_v)
    pltpu.sync_copy(dst_hbm.at[wid], dst_v)
    r0 = s * _RPT
    pltpu.sync_copy(
        g_hbm.at[pl.ds(r0, _RPT), pl.ds(col0, _H)], acc_sh.at[pl.ds(r0, _RPT)]
    )
    plsc.subcore_barrier()

    def gathers(h, grp, sem):
        for k in range(_K):
            pltpu.async_copy(g_hbm.at[src_v.at[grp * _K + k]], bufs.at[h, k], sem)

    def wait_gathers(h, grp, sem):
        for k in range(_K):
            pltpu.make_async_copy(
                g_hbm.at[src_v.at[grp * _K + k]], bufs.at[h, k], sem
            ).wait()

    def halve(h):
        for k in range(_K):

            def hrow(i, carry):
                for r16 in range(16):
                    for j4 in range(_H // 16):
                        hbufs[h, k, i * 16 + r16, pl.ds(16 * j4, 16)] = bufs[
                            h, k, i * 16 + r16, pl.ds(16 * j4, 16)
                        ]
                return carry

            lax.fori_loop(0, _CHUNK // 16, hrow, 0)

    def scatters(h, grp, sem):
        for k in range(_K):
            pltpu.async_copy(
                hbufs.at[h, k], acc_sh.at[dst_v.at[grp * _K + k]], sem, add=True
            )

    def wait_scatters(h, grp, sem):
        for k in range(_K):
            pltpu.make_async_copy(
                hbufs.at[h, k], acc_sh.at[dst_v.at[grp * _K + k]], sem
            ).wait()

    gathers(0, 0, gsem_a)

    def body(j, carry):
        ga = 2 * j
        gb = 2 * j + 1
        wait_gathers(0, ga, gsem_a)
        halve(0)
        scatters(0, ga, ssem_a)

        @pl.when(j > 0)
        def _():
            wait_scatters(1, gb - 2, ssem_b)

        gathers(1, gb, gsem_b)
        wait_gathers(1, gb, gsem_b)
        halve(1)
        scatters(1, gb, ssem_b)

        @pl.when(j < _NG // 2 - 1)
        def _():
            wait_scatters(0, ga, ssem_a)
            gathers(0, ga + 2, gsem_a)

        return carry

    lax.fori_loop(0, _NG // 2, body, 0)
    wait_scatters(0, _NG - 2, ssem_a)
    wait_scatters(1, _NG - 1, ssem_b)
    plsc.subcore_barrier()
    pltpu.sync_copy(
        acc_sh.at[pl.ds(r0, _RPT)],
        out_hbm.at[c].at[pl.ds(r0, _RPT), pl.ds(0, _H)],
    )


_BLK = 512
_GRID = _NPAD // _BLK


def _dis_blk(c0, c1, i):
    deg = c0[:, 0:1] + c1[:, 0:1]
    rows = i * _BLK + lax.broadcasted_iota(jnp.int32, (_BLK, 1), 0)
    return jnp.where(rows < _N, lax.rsqrt(deg), 0.0)


def _wide(v):
    return jnp.concatenate([v, jnp.zeros((_BLK, _W128 - _H), jnp.float32)], axis=1)


def _bspec():
    return pl.BlockSpec((_BLK, _W128), lambda i: (i, 0))


def _tc_prologue_body(x_ref, w_ref, c0_ref, c1_ref, o_ref):
    i = pl.program_id(0)
    dis = _dis_blk(c0_ref[...], c1_ref[...], i)
    o_ref[...] = _wide(
        dis * jnp.dot(x_ref[...], w_ref[...], preferred_element_type=jnp.float32)
    )


_tc_prologue = pl.pallas_call(
    _tc_prologue_body,
    out_shape=jax.ShapeDtypeStruct((_NPAD, _W128), jnp.float32),
    grid=(_GRID,),
    in_specs=[
        pl.BlockSpec((_BLK, _D), lambda i: (i, 0)),
        pl.BlockSpec((_D, _H), lambda i: (0, 0)),
        _bspec(),
        _bspec(),
    ],
    out_specs=_bspec(),
)


def _tc_fuse_body(a0_ref, a1_ref, c0_ref, c1_ref, b_ref, w_ref, o_ref):
    i = pl.program_id(0)
    dis = _dis_blk(c0_ref[...], c1_ref[...], i)
    am = (a0_ref[...] + a1_ref[...])[:, :_H]
    h = dis * am + b_ref[...]
    h = jnp.maximum(h, 0.0)
    o_ref[...] = _wide(
        dis * jnp.dot(h, w_ref[...], preferred_element_type=jnp.float32)
    )


_tc_fuse = pl.pallas_call(
    _tc_fuse_body,
    out_shape=jax.ShapeDtypeStruct((_NPAD, _W128), jnp.float32),
    grid=(_GRID,),
    in_specs=[
        _bspec(),
        _bspec(),
        _bspec(),
        _bspec(),
        pl.BlockSpec((1, _H), lambda i: (0, 0)),
        pl.BlockSpec((_H, _H), lambda i: (0, 0)),
    ],
    out_specs=_bspec(),
)


def _tc_final_body(a0_ref, a1_ref, c0_ref, c1_ref, b_ref, w_ref, bo_ref, o_ref):
    i = pl.program_id(0)
    dis = _dis_blk(c0_ref[...], c1_ref[...], i)
    am = (a0_ref[...] + a1_ref[...])[:, :_H]
    h = dis * am + b_ref[...]
    o_ref[...] = (
        jnp.dot(h, w_ref[...], preferred_element_type=jnp.float32) + bo_ref[...]
    )


_tc_final = pl.pallas_call(
    _tc_final_body,
    out_shape=jax.ShapeDtypeStruct((_NPAD, _D), jnp.float32),
    grid=(_GRID,),
    in_specs=[
        _bspec(),
        _bspec(),
        _bspec(),
        _bspec(),
        pl.BlockSpec((1, _H), lambda i: (0, 0)),
        pl.BlockSpec((_H, _D), lambda i: (0, 0)),
        pl.BlockSpec((1, _D), lambda i: (0, 0)),
    ],
    out_specs=pl.BlockSpec((_BLK, _D), lambda i: (i, 0)),
)


def kernel(x, edge_index, W0, b0, W1, b1, W2, b2, W3, b3, Wout, bout):
    src = edge_index[0].astype(jnp.int32)
    dst = edge_index[1].astype(jnp.int32)
    npad = _EPAD - _E
    srcp = jnp.concatenate([src, jnp.full((npad,), _N, jnp.int32)]).reshape(
        _NW, _NCH, _CHUNK
    )
    dstp = jnp.concatenate([dst, jnp.full((npad,), _N, jnp.int32)]).reshape(
        _NW, _NCH, _CHUNK
    )
    xp = jnp.pad(x, ((0, _NPAD - _N), (0, 0)))

    ones_wide = jnp.concatenate(
        [jnp.ones((_NPAD, _H), jnp.float32), jnp.zeros((_NPAD, _H), jnp.float32)],
        axis=1,
    )
    cacc = _sc_agg(ones_wide, srcp, dstp)
    c0, c1 = cacc[0], cacc[1]

    g = _tc_prologue(xp, W0, c0, c1)
    for bb, ww in ((b0, W1), (b1, W2), (b2, W3)):
        acc = _sc_agg(g, srcp, dstp)
        g = _tc_fuse(acc[0], acc[1], c0, c1, bb.reshape(1, _H), ww)
    acc = _sc_agg(g, srcp, dstp)
    out = _tc_final(
        acc[0], acc[1], c0, c1, b3.reshape(1, _H), Wout, bout.reshape(1, _D)
    )
    return out[:_N]


# speedup vs baseline: 2.7753x; 2.7381x over previous
import functools

import jax
import jax.numpy as jnp
from jax import lax
from jax.experimental import pallas as pl
from jax.experimental.pallas import tpu as pltpu
from jax.experimental.pallas import tpu_sc as plsc

_N = 10000
_E = 320000
_D = 128
_H = 64

_NC = 2
_NS = 16
_NW = _NC * _NS

_RPT = 640
_NPAD = _NS * _RPT
_CHUNK = 128
_EPT = 10240
_NCH = _EPT // _CHUNK
_EPAD = _NW * _EPT
_DEGW = 16

_mesh = plsc.VectorSubcoreMesh(
    core_axis_name="c", subcore_axis_name="s", num_cores=_NC, num_subcores=_NS
)


@functools.partial(
    pl.kernel,
    out_type=jax.ShapeDtypeStruct((_NC, _NPAD, _DEGW), jnp.float32),
    mesh=_mesh,
    scratch_types=[
        pltpu.VMEM((_NCH, _CHUNK), jnp.int32),
        pltpu.VMEM((_CHUNK, _DEGW), jnp.float32),
        pltpu.VMEM((_CHUNK, _DEGW), jnp.float32),
        pltpu.VMEM_SHARED((_NPAD, _DEGW), jnp.float32),
    ],
    compiler_params=pltpu.CompilerParams(use_tc_tiling_on_sc=False),
)
def _sc_deg(dst_hbm, out_hbm, dst_v, ones_v, zeros_v, dacc_sh):
    c = lax.axis_index("c")
    s = lax.axis_index("s")
    wid = c * _NS + s
    pltpu.sync_copy(dst_hbm.at[wid], dst_v)

    def fill(i, carry):
        ones_v[i, :] = jnp.full((_DEGW,), 1.0, jnp.float32)
        zeros_v[i, :] = jnp.zeros((_DEGW,), jnp.float32)
        return carry

    lax.fori_loop(0, _CHUNK, fill, 0)
    r0 = s * _RPT
    for k in range(_RPT // _CHUNK):
        pltpu.sync_copy(zeros_v, dacc_sh.at[pl.ds(r0 + k * _CHUNK, _CHUNK)])
    plsc.subcore_barrier()

    def body(j, carry):
        pltpu.sync_copy(ones_v, dacc_sh.at[dst_v.at[j]], add=True)
        return carry

    lax.fori_loop(0, _NCH, body, 0)
    plsc.subcore_barrier()
    pltpu.sync_copy(dacc_sh.at[pl.ds(r0, _RPT)], out_hbm.at[c, pl.ds(r0, _RPT)])


_K = 4
_NG = _NCH // _K


@functools.partial(
    pl.kernel,
    out_type=jax.ShapeDtypeStruct((_NC, _NPAD, _H), jnp.float32),
    mesh=_mesh,
    scratch_types=[
        pltpu.VMEM((_NCH, _CHUNK), jnp.int32),
        pltpu.VMEM((_NCH, _CHUNK), jnp.int32),
        pltpu.VMEM((2, _K, _CHUNK, _H), jnp.float32),
        pltpu.VMEM_SHARED((_NPAD, _H), jnp.float32),
        pltpu.SemaphoreType.DMA,
        pltpu.SemaphoreType.DMA,
        pltpu.SemaphoreType.DMA,
        pltpu.SemaphoreType.DMA,
    ],
    compiler_params=pltpu.CompilerParams(use_tc_tiling_on_sc=False),
)
def _sc_agg(
    g_hbm, src_hbm, dst_hbm, out_hbm, src_v, dst_v, bufs, acc_sh,
    gsem_a, gsem_b, ssem_a, ssem_b,
):
    c = lax.axis_index("c")
    s = lax.axis_index("s")
    wid = c * _NS + s
    pltpu.sync_copy(src_hbm.at[wid], src_v)
    pltpu.sync_copy(dst_hbm.at[wid], dst_v)
    r0 = s * _RPT
    pltpu.sync_copy(g_hbm.at[pl.ds(r0, _RPT)], acc_sh.at[pl.ds(r0, _RPT)])
    plsc.subcore_barrier()

    def gathers(h, grp, sem):
        for k in range(_K):
            pltpu.async_copy(g_hbm.at[src_v.at[grp * _K + k]], bufs.at[h, k], sem)

    def wait_gathers(h, grp, sem):
        for k in range(_K):
            pltpu.make_async_copy(
                g_hbm.at[src_v.at[grp * _K + k]], bufs.at[h, k], sem
            ).wait()

    def scatters(h, grp, sem):
        for k in range(_K):
            pltpu.async_copy(
                bufs.at[h, k], acc_sh.at[dst_v.at[grp * _K + k]], sem, add=True
            )

    def wait_scatters(h, grp, sem):
        for k in range(_K):
            pltpu.make_async_copy(
                bufs.at[h, k], acc_sh.at[dst_v.at[grp * _K + k]], sem
            ).wait()

    gathers(0, 0, gsem_a)

    def body(j, carry):
        ga = 2 * j
        gb = 2 * j + 1
        wait_gathers(0, ga, gsem_a)
        scatters(0, ga, ssem_a)

        @pl.when(j > 0)
        def _():
            wait_scatters(1, gb - 2, ssem_b)

        gathers(1, gb, gsem_b)
        wait_gathers(1, gb, gsem_b)
        scatters(1, gb, ssem_b)

        @pl.when(j < _NG // 2 - 1)
        def _():
            wait_scatters(0, ga, ssem_a)
            gathers(0, ga + 2, gsem_a)

        return carry

    lax.fori_loop(0, _NG // 2, body, 0)
    wait_scatters(0, _NG - 2, ssem_a)
    wait_scatters(1, _NG - 1, ssem_b)
    plsc.subcore_barrier()
    pltpu.sync_copy(acc_sh.at[pl.ds(r0, _RPT)], out_hbm.at[c, pl.ds(r0, _RPT)])


_BLK = 512
_GRID = _NPAD // _BLK


def _dis_blk(d0, d1, i):
    deg = d0[:, 0:1] + d1[:, 0:1] + 1.0
    rows = i * _BLK + lax.broadcasted_iota(jnp.int32, (_BLK, 1), 0)
    return jnp.where(rows < _N, lax.rsqrt(deg), 0.0)


def _tc_prologue_body(x_ref, w_ref, d0_ref, d1_ref, o_ref):
    i = pl.program_id(0)
    dis = _dis_blk(d0_ref[...], d1_ref[...], i)
    o_ref[...] = dis * jnp.dot(
        x_ref[...], w_ref[...], preferred_element_type=jnp.float32
    )


_tc_prologue = pl.pallas_call(
    _tc_prologue_body,
    out_shape=jax.ShapeDtypeStruct((_NPAD, _H), jnp.float32),
    grid=(_GRID,),
    in_specs=[
        pl.BlockSpec((_BLK, _D), lambda i: (i, 0)),
        pl.BlockSpec((_D, _H), lambda i: (0, 0)),
        pl.BlockSpec((_BLK, _DEGW), lambda i: (i, 0)),
        pl.BlockSpec((_BLK, _DEGW), lambda i: (i, 0)),
    ],
    out_specs=pl.BlockSpec((_BLK, _H), lambda i: (i, 0)),
)


def _tc_fuse_body(a0_ref, a1_ref, g_ref, d0_ref, d1_ref, b_ref, w_ref, o_ref):
    i = pl.program_id(0)
    dis = _dis_blk(d0_ref[...], d1_ref[...], i)
    h = dis * (a0_ref[...] + a1_ref[...] - g_ref[...]) + b_ref[...]
    h = jnp.maximum(h, 0.0)
    o_ref[...] = dis * jnp.dot(h, w_ref[...], preferred_element_type=jnp.float32)


_tc_fuse = pl.pallas_call(
    _tc_fuse_body,
    out_shape=jax.ShapeDtypeStruct((_NPAD, _H), jnp.float32),
    grid=(_GRID,),
    in_specs=[
        pl.BlockSpec((_BLK, _H), lambda i: (i, 0)),
        pl.BlockSpec((_BLK, _H), lambda i: (i, 0)),
        pl.BlockSpec((_BLK, _H), lambda i: (i, 0)),
        pl.BlockSpec((_BLK, _DEGW), lambda i: (i, 0)),
        pl.BlockSpec((_BLK, _DEGW), lambda i: (i, 0)),
        pl.BlockSpec((1, _H), lambda i: (0, 0)),
        pl.BlockSpec((_H, _H), lambda i: (0, 0)),
    ],
    out_specs=pl.BlockSpec((_BLK, _H), lambda i: (i, 0)),
)


def _tc_final_body(a0_ref, a1_ref, g_ref, d0_ref, d1_ref, b_ref, w_ref, bo_ref, o_ref):
    i = pl.program_id(0)
    dis = _dis_blk(d0_ref[...], d1_ref[...], i)
    h = dis * (a0_ref[...] + a1_ref[...] - g_ref[...]) + b_ref[...]
    o_ref[...] = (
        jnp.dot(h, w_ref[...], preferred_element_type=jnp.float32) + bo_ref[...]
    )


_tc_final = pl.pallas_call(
    _tc_final_body,
    out_shape=jax.ShapeDtypeStruct((_NPAD, _D), jnp.float32),
    grid=(_GRID,),
    in_specs=[
        pl.BlockSpec((_BLK, _H), lambda i: (i, 0)),
        pl.BlockSpec((_BLK, _H), lambda i: (i, 0)),
        pl.BlockSpec((_BLK, _H), lambda i: (i, 0)),
        pl.BlockSpec((_BLK, _DEGW), lambda i: (i, 0)),
        pl.BlockSpec((_BLK, _DEGW), lambda i: (i, 0)),
        pl.BlockSpec((1, _H), lambda i: (0, 0)),
        pl.BlockSpec((_H, _D), lambda i: (0, 0)),
        pl.BlockSpec((1, _D), lambda i: (0, 0)),
    ],
    out_specs=pl.BlockSpec((_BLK, _D), lambda i: (i, 0)),
)


def kernel(x, edge_index, W0, b0, W1, b1, W2, b2, W3, b3, Wout, bout):
    src = edge_index[0].astype(jnp.int32)
    dst = edge_index[1].astype(jnp.int32)
    npad = _EPAD - _E
    srcp = jnp.concatenate([src, jnp.full((npad,), _N, jnp.int32)]).reshape(
        _NW, _NCH, _CHUNK
    )
    dstp = jnp.concatenate([dst, jnp.full((npad,), _N, jnp.int32)]).reshape(
        _NW, _NCH, _CHUNK
    )
    xp = jnp.pad(x, ((0, _NPAD - _N), (0, 0)))

    degp = _sc_deg(dstp)
    d0, d1 = degp[0], degp[1]

    g = _tc_prologue(xp, W0, d0, d1)
    for bb, ww in ((b0, W1), (b1, W2), (b2, W3)):
        acc = _sc_agg(g, srcp, dstp)
        g = _tc_fuse(acc[0], acc[1], g, d0, d1, bb.reshape(1, _H), ww)
    acc = _sc_agg(g, srcp, dstp)
    out = _tc_final(
        acc[0], acc[1], g, d0, d1, b3.reshape(1, _H), Wout, bout.reshape(1, _D)
    )
    return out[:_N]


# speedup vs baseline: 2.8427x; 1.0243x over previous
import functools

import jax
import jax.numpy as jnp
from jax import lax
from jax.experimental import pallas as pl
from jax.experimental.pallas import tpu as pltpu
from jax.experimental.pallas import tpu_sc as plsc

_N = 10000
_E = 320000
_D = 128
_H = 64

_NC = 2
_NS = 16
_NW = _NC * _NS

_RPT = 640
_NPAD = _NS * _RPT
_CHUNK = 128
_EPT = 10240
_NCH = _EPT // _CHUNK
_EPAD = _NW * _EPT
_DEGW = 16

_mesh = plsc.VectorSubcoreMesh(
    core_axis_name="c", subcore_axis_name="s", num_cores=_NC, num_subcores=_NS
)


@functools.partial(
    pl.kernel,
    out_type=jax.ShapeDtypeStruct((_NC, _NPAD, 128), jnp.float32),
    mesh=_mesh,
    scratch_types=[
        pltpu.VMEM((_NCH, _CHUNK), jnp.int32),
        pltpu.VMEM((_CHUNK, _DEGW), jnp.float32),
        pltpu.VMEM((_CHUNK, _DEGW), jnp.float32),
        pltpu.VMEM_SHARED((_NPAD, _DEGW), jnp.float32),
    ],
    compiler_params=pltpu.CompilerParams(use_tc_tiling_on_sc=False),
)
def _sc_deg(dst_hbm, out_hbm, dst_v, ones_v, zeros_v, dacc_sh):
    c = lax.axis_index("c")
    s = lax.axis_index("s")
    wid = c * _NS + s
    pltpu.sync_copy(dst_hbm.at[wid], dst_v)

    def fill(i, carry):
        ones_v[i, :] = jnp.full((_DEGW,), 1.0, jnp.float32)
        zeros_v[i, :] = jnp.zeros((_DEGW,), jnp.float32)
        return carry

    lax.fori_loop(0, _CHUNK, fill, 0)
    r0 = s * _RPT
    for k in range(_RPT // _CHUNK):
        pltpu.sync_copy(zeros_v, dacc_sh.at[pl.ds(r0 + k * _CHUNK, _CHUNK)])
    plsc.subcore_barrier()

    def body(j, carry):
        pltpu.sync_copy(ones_v, dacc_sh.at[dst_v.at[j]], add=True)
        return carry

    lax.fori_loop(0, _NCH, body, 0)
    plsc.subcore_barrier()
    pltpu.sync_copy(
        dacc_sh.at[pl.ds(r0, _RPT)],
        out_hbm.at[c].at[pl.ds(r0, _RPT), pl.ds(0, _DEGW)],
    )


_K = 4
_NG = _NCH // _K


@functools.partial(
    pl.kernel,
    out_type=jax.ShapeDtypeStruct((_NC, _NPAD, 128), jnp.float32),
    mesh=_mesh,
    scratch_types=[
        pltpu.VMEM((_NCH, _CHUNK), jnp.int32),
        pltpu.VMEM((_NCH, _CHUNK), jnp.int32),
        pltpu.VMEM((2, _K, _CHUNK, _H), jnp.float32),
        pltpu.VMEM_SHARED((_NPAD, _H), jnp.float32),
        pltpu.SemaphoreType.DMA,
        pltpu.SemaphoreType.DMA,
        pltpu.SemaphoreType.DMA,
        pltpu.SemaphoreType.DMA,
    ],
    compiler_params=pltpu.CompilerParams(use_tc_tiling_on_sc=False),
)
def _sc_agg(
    g_hbm, src_hbm, dst_hbm, out_hbm, src_v, dst_v, bufs, acc_sh,
    gsem_a, gsem_b, ssem_a, ssem_b,
):
    c = lax.axis_index("c")
    s = lax.axis_index("s")
    wid = c * _NS + s
    pltpu.sync_copy(src_hbm.at[wid], src_v)
    pltpu.sync_copy(dst_hbm.at[wid], dst_v)
    r0 = s * _RPT
    pltpu.sync_copy(g_hbm.at[pl.ds(r0, _RPT)], acc_sh.at[pl.ds(r0, _RPT)])
    plsc.subcore_barrier()

    def gathers(h, grp, sem):
        for k in range(_K):
            pltpu.async_copy(g_hbm.at[src_v.at[grp * _K + k]], bufs.at[h, k], sem)

    def wait_gathers(h, grp, sem):
        for k in range(_K):
            pltpu.make_async_copy(
                g_hbm.at[src_v.at[grp * _K + k]], bufs.at[h, k], sem
            ).wait()

    def scatters(h, grp, sem):
        for k in range(_K):
            pltpu.async_copy(
                bufs.at[h, k], acc_sh.at[dst_v.at[grp * _K + k]], sem, add=True
            )

    def wait_scatters(h, grp, sem):
        for k in range(_K):
            pltpu.make_async_copy(
                bufs.at[h, k], acc_sh.at[dst_v.at[grp * _K + k]], sem
            ).wait()

    gathers(0, 0, gsem_a)

    def body(j, carry):
        ga = 2 * j
        gb = 2 * j + 1
        wait_gathers(0, ga, gsem_a)
        scatters(0, ga, ssem_a)

        @pl.when(j > 0)
        def _():
            wait_scatters(1, gb - 2, ssem_b)

        gathers(1, gb, gsem_b)
        wait_gathers(1, gb, gsem_b)
        scatters(1, gb, ssem_b)

        @pl.when(j < _NG // 2 - 1)
        def _():
            wait_scatters(0, ga, ssem_a)
            gathers(0, ga + 2, gsem_a)

        return carry

    lax.fori_loop(0, _NG // 2, body, 0)
    wait_scatters(0, _NG - 2, ssem_a)
    wait_scatters(1, _NG - 1, ssem_b)
    plsc.subcore_barrier()
    pltpu.sync_copy(
        acc_sh.at[pl.ds(r0, _RPT)],
        out_hbm.at[c].at[pl.ds(r0, _RPT), pl.ds(0, _H)],
    )


_BLK = 512
_GRID = _NPAD // _BLK


def _dis_blk(d0, d1, i):
    deg = d0[:, 0:1] + d1[:, 0:1] + 1.0
    rows = i * _BLK + lax.broadcasted_iota(jnp.int32, (_BLK, 1), 0)
    return jnp.where(rows < _N, lax.rsqrt(deg), 0.0)


def _tc_prologue_body(x_ref, w_ref, d0_ref, d1_ref, o_ref):
    i = pl.program_id(0)
    dis = _dis_blk(d0_ref[...], d1_ref[...], i)
    o_ref[...] = dis * jnp.dot(
        x_ref[...], w_ref[...], preferred_element_type=jnp.float32
    )


_tc_prologue = pl.pallas_call(
    _tc_prologue_body,
    out_shape=jax.ShapeDtypeStruct((_NPAD, _H), jnp.float32),
    grid=(_GRID,),
    in_specs=[
        pl.BlockSpec((_BLK, _D), lambda i: (i, 0)),
        pl.BlockSpec((_D, _H), lambda i: (0, 0)),
        pl.BlockSpec((_BLK, 128), lambda i: (i, 0)),
        pl.BlockSpec((_BLK, 128), lambda i: (i, 0)),
    ],
    out_specs=pl.BlockSpec((_BLK, _H), lambda i: (i, 0)),
)


def _tc_fuse_body(a0_ref, a1_ref, g_ref, d0_ref, d1_ref, b_ref, w_ref, o_ref):
    i = pl.program_id(0)
    dis = _dis_blk(d0_ref[...], d1_ref[...], i)
    h = dis * ((a0_ref[...] + a1_ref[...])[:, :_H] - g_ref[...]) + b_ref[...]
    h = jnp.maximum(h, 0.0)
    o_ref[...] = dis * jnp.dot(h, w_ref[...], preferred_element_type=jnp.float32)


_tc_fuse = pl.pallas_call(
    _tc_fuse_body,
    out_shape=jax.ShapeDtypeStruct((_NPAD, _H), jnp.float32),
    grid=(_GRID,),
    in_specs=[
        pl.BlockSpec((_BLK, 128), lambda i: (i, 0)),
        pl.BlockSpec((_BLK, 128), lambda i: (i, 0)),
        pl.BlockSpec((_BLK, _H), lambda i: (i, 0)),
        pl.BlockSpec((_BLK, 128), lambda i: (i, 0)),
        pl.BlockSpec((_BLK, 128), lambda i: (i, 0)),
        pl.BlockSpec((1, _H), lambda i: (0, 0)),
        pl.BlockSpec((_H, _H), lambda i: (0, 0)),
    ],
    out_specs=pl.BlockSpec((_BLK, _H), lambda i: (i, 0)),
)


def _tc_final_body(a0_ref, a1_ref, g_ref, d0_ref, d1_ref, b_ref, w_ref, bo_ref, o_ref):
    i = pl.program_id(0)
    dis = _dis_blk(d0_ref[...], d1_ref[...], i)
    h = dis * ((a0_ref[...] + a1_ref[...])[:, :_H] - g_ref[...]) + b_ref[...]
    o_ref[...] = (
        jnp.dot(h, w_ref[...], preferred_element_type=jnp.float32) + bo_ref[...]
    )


_tc_final = pl.pallas_call(
    _tc_final_body,
    out_shape=jax.ShapeDtypeStruct((_NPAD, _D), jnp.float32),
    grid=(_GRID,),
    in_specs=[
        pl.BlockSpec((_BLK, 128), lambda i: (i, 0)),
        pl.BlockSpec((_BLK, 128), lambda i: (i, 0)),
        pl.BlockSpec((_BLK, _H), lambda i: (i, 0)),
        pl.BlockSpec((_BLK, 128), lambda i: (i, 0)),
        pl.BlockSpec((_BLK, 128), lambda i: (i, 0)),
        pl.BlockSpec((1, _H), lambda i: (0, 0)),
        pl.BlockSpec((_H, _D), lambda i: (0, 0)),
        pl.BlockSpec((1, _D), lambda i: (0, 0)),
    ],
    out_specs=pl.BlockSpec((_BLK, _D), lambda i: (i, 0)),
)


def kernel(x, edge_index, W0, b0, W1, b1, W2, b2, W3, b3, Wout, bout):
    src = edge_index[0].astype(jnp.int32)
    dst = edge_index[1].astype(jnp.int32)
    npad = _EPAD - _E
    srcp = jnp.concatenate([src, jnp.full((npad,), _N, jnp.int32)]).reshape(
        _NW, _NCH, _CHUNK
    )
    dstp = jnp.concatenate([dst, jnp.full((npad,), _N, jnp.int32)]).reshape(
        _NW, _NCH, _CHUNK
    )
    xp = jnp.pad(x, ((0, _NPAD - _N), (0, 0)))

    degp = _sc_deg(dstp)
    d0, d1 = degp[0], degp[1]

    g = _tc_prologue(xp, W0, d0, d1)
    for bb, ww in ((b0, W1), (b1, W2), (b2, W3)):
        acc = _sc_agg(g, srcp, dstp)
        g = _tc_fuse(acc[0], acc[1], g, d0, d1, bb.reshape(1, _H), ww)
    acc = _sc_agg(g, srcp, dstp)
    out = _tc_final(
        acc[0], acc[1], g, d0, d1, b3.reshape(1, _H), Wout, bout.reshape(1, _D)
    )
    return out[:_N]
